# baseline (speedup 1.0000x reference)
import functools

import jax
import jax.numpy as jnp
from jax import lax
from jax.experimental import pallas as pl
from jax.experimental.pallas import tpu as pltpu
from jax.experimental.pallas import tpu_sc as plsc

K_TOP = 32
_L = 16
_BLK = 128
_NC = 2
_NS = 16
_ECAP = 64
_EBUF = _ECAP + _BLK + _L


def _enc_body(x_ref, w_ref, b_ref, out_ref, bm_ref, *, B, BH):
    acc = jax.lax.dot_general(
        x_ref[...], w_ref[...], (((1,), (1,)), ((), ())),
        preferred_element_type=jnp.float32)
    out = jnp.maximum(acc + b_ref[0, :][None, :], 0.0)
    out_ref[...] = out
    nb = BH // _BLK
    parts = [jnp.max(out[:, g * _BLK:(g + 1) * _BLK], axis=1, keepdims=True)
             for g in range(nb)]
    bm_ref[0, :, :] = jnp.concatenate(parts, axis=1)


def _sc_body(pre_hbm, bm_hbm, wd_hbm, thr_hbm, cut_hbm, dx_hbm,
             row_v, bm3_v, bm_v, t0_v, ev_v, ei_v, idx_v, col_v, acc_v,
             thr_s, cut_s, sem, gsem, *, H, D, k, nb, num_cores):
    wid = lax.axis_index("s") * num_cores + lax.axis_index("c")
    cp = pltpu.async_copy(pre_hbm.at[wid], row_v, sem)
    pltpu.sync_copy(bm_hbm, bm3_v)

    nblk = H // _BLK
    nbv = nblk // _L
    ngrp = _BLK // _L
    iota16 = lax.iota(jnp.int32, _L)
    big = jnp.int32(1 << 30)
    neg = jnp.full((_L,), -1.0, jnp.float32)
    lane0 = iota16 == 0
    wid_v = jnp.full((_L,), wid, jnp.int32)

    for v in range(nbv):
        p = v * _L + iota16
        vals = plsc.load_gather(bm3_v, [p // nb, wid_v, p % nb])
        bm_v[pl.ds(v * _L, _L)] = vals

    def _perm(v, perm):
        return v.at[perm].get(mode="promise_in_bounds")

    def _bfly(v, op):
        for s in (8, 4, 2, 1):
            v = op(v, _perm(v, iota16 ^ s))
        return v

    def _scan_bm(ref):
        bv = ref[pl.ds(0, _L)]
        bb = iota16
        for g in range(1, nbv):
            v = ref[pl.ds(g * _L, _L)]
            b = g * _L + iota16
            take = (v > bv) | ((v == bv) & (b < bb))
            bv = jnp.where(take, v, bv)
            bb = jnp.where(take, b, bb)
        m = _bfly(bv, jnp.maximum)
        p = _bfly(jnp.where(bv == m, bb, big), jnp.minimum)
        return m, p

    for v in range(nbv):
        t0_v[pl.ds(v * _L, _L)] = bm_v[pl.ds(v * _L, _L)]

    def rm(i, c):
        m, p = _scan_bm(t0_v)
        plsc.store_scatter(t0_v, [p], neg, mask=lane0)
        return c

    lax.fori_loop(0, k - 1, rm, 0)
    t0 = t0_v[pl.ds(0, _L)]
    for g in range(1, nbv):
        t0 = jnp.maximum(t0, t0_v[pl.ds(g * _L, _L)])
    t0 = _bfly(t0, jnp.maximum)
    t0_scalar = jnp.max(t0)

    cp.wait()

    def emit(t, carry):
        m, p = _scan_bm(bm_v)
        base = p * _BLK
        gv = gi = None
        regs = []
        for g in range(ngrp):
            ii = base + g * _L + iota16
            vv = plsc.load_gather(row_v, [ii])
            regs.append((vv, ii))
            if gv is None:
                gv, gi = vv, ii
            else:
                take = (vv > gv) | ((vv == gv) & (ii < gi))
                gv = jnp.where(take, vv, gv)
                gi = jnp.where(take, ii, gi)
        mv = _bfly(gv, jnp.maximum)
        mi = _bfly(jnp.where(gv == mv, gi, big), jnp.minimum)
        plsc.store_scatter(row_v, [mi], neg, mask=lane0)
        nm = None
        for vv, ii in regs:
            vv = jnp.where(ii == mi, -1.0, vv)
            nm = vv if nm is None else jnp.maximum(nm, vv)
        nm = _bfly(nm, jnp.maximum)
        plsc.store_scatter(bm_v, [p], nm, mask=lane0)
        return (mv, mi)

    mv, mi = lax.fori_loop(0, k, emit, (neg, iota16))

    def _rem_max():
        v = bm_v[pl.ds(0, _L)]
        for g in range(1, nbv):
            v = jnp.maximum(v, bm_v[pl.ds(g * _L, _L)])
        return jnp.max(v)

    def wcond(st):
        go, off = st
        return go & (off <= _ECAP)

    def wbody(st):
        _, off = st
        m, p = _scan_bm(bm_v)
        base = p * _BLK
        nm = None
        for g in range(ngrp):
            ii = base + g * _L + iota16
            vv = plsc.load_gather(row_v, [ii])
            msk = vv >= t0
            c = jnp.sum(jnp.where(msk, 1, 0))
            plsc.store_compressed(ev_v.at[pl.ds(off, _L)], vv, mask=msk)
            plsc.store_compressed(ei_v.at[pl.ds(off, _L)], ii, mask=msk)
            plsc.store_scatter(row_v, [ii], neg, mask=msk)
            off = off + c
            vv = jnp.where(msk, -1.0, vv)
            nm = vv if nm is None else jnp.maximum(nm, vv)
        nm = _bfly(nm, jnp.maximum)
        plsc.store_scatter(bm_v, [p], nm, mask=lane0)
        return (_rem_max() >= t0_scalar, off)

    go0 = _rem_max() >= t0_scalar
    go, ne = lax.while_loop(wcond, wbody, (go0, jnp.int32(0)))
    overflow = jnp.where(go, jnp.int32(1), jnp.int32(0))

    nv_d = D // _L

    def zb(i, c):
        acc_v[pl.ds(i * _L, _L)] = jnp.zeros((_L,), jnp.float32)
        return c

    lax.fori_loop(0, nv_d, zb, 0)

    def gather_one(e, c):
        j = plsc.load_gather(ei_v, [jnp.minimum(
            jnp.full((_L,), e, jnp.int32), _EBUF - 1)])
        val = plsc.load_gather(ev_v, [jnp.minimum(
            jnp.full((_L,), e, jnp.int32), _EBUF - 1)])

        def ib(i, c2):
            idx_v[pl.ds(i * _L, _L)] = j + H * (i * _L + iota16)
            return c2

        lax.fori_loop(0, nv_d, ib, 0)
        cps = [pltpu.async_copy(
            wd_hbm.at[idx_v.at[pl.ds(g * _BLK, _BLK)]],
            col_v.at[pl.ds(g * _BLK, _BLK)], gsem)
            for g in range(D // _BLK)]
        for c2 in cps:
            c2.wait()

        def ab(i, c2):
            acc_v[pl.ds(i * _L, _L)] = (acc_v[pl.ds(i * _L, _L)]
                                        - val * col_v[pl.ds(i * _L, _L)])
            return c2

        lax.fori_loop(0, nv_d, ab, 0)
        return c

    ngather = jnp.where(overflow > 0, jnp.int32(0), ne)
    lax.fori_loop(0, ngather, gather_one, 0)

    thr_s[...] = mv
    ov_v = jnp.full((_L,), overflow, jnp.int32)
    cut_s[...] = jnp.where(lane0, mi, jnp.where(iota16 == 1, ov_v, 0))
    pltpu.sync_copy(thr_s, thr_hbm.at[wid])
    pltpu.sync_copy(cut_s, cut_hbm.at[wid])
    pltpu.sync_copy(acc_v, dx_hbm.at[wid])


def _sc_stage(pre, bm, wd_flat, *, B, H, D, k):
    mesh = plsc.VectorSubcoreMesh(core_axis_name="c", subcore_axis_name="s",
                                  num_cores=_NC, num_subcores=_NS)
    nsteps, _, nb = bm.shape
    return pl.kernel(
        functools.partial(_sc_body, H=H, D=D, k=k, nb=nb, num_cores=_NC),
        out_type=(jax.ShapeDtypeStruct((B, _L), jnp.float32),
                  jax.ShapeDtypeStruct((B, _L), jnp.int32),
                  jax.ShapeDtypeStruct((B, D), jnp.float32)),
        mesh=mesh,
        compiler_params=pltpu.CompilerParams(needs_layout_passes=False),
        scratch_types=[
            pltpu.VMEM((H,), jnp.float32),
            pltpu.VMEM((nsteps, B, nb), jnp.float32),
            pltpu.VMEM((H // _BLK,), jnp.float32),
            pltpu.VMEM((H // _BLK,), jnp.float32),
            pltpu.VMEM((_EBUF,), jnp.float32),
            pltpu.VMEM((_EBUF,), jnp.int32),
            pltpu.VMEM((D,), jnp.int32),
            pltpu.VMEM((D,), jnp.float32),
            pltpu.VMEM((D,), jnp.float32),
            pltpu.VMEM((_L,), jnp.float32),
            pltpu.VMEM((_L,), jnp.int32),
            pltpu.SemaphoreType.DMA,
            pltpu.SemaphoreType.DMA,
        ],
    )(pre, bm, wd_flat)


def _dec_guess_body(p_ref, w_ref, b_ref, bm_ref, out_ref, t0_ref,
                    *, B, D, BH, k, nsteps):
    @pl.when(pl.program_id(0) == 0)
    def _():
        bmr = jnp.concatenate([bm_ref[j] for j in range(nsteps)], axis=1)
        nblk = bmr.shape[1]
        col = jax.lax.broadcasted_iota(jnp.int32, (B, nblk), 1)

        def rm(i, b):
            m = jnp.max(b, axis=1, keepdims=True)
            idx = jnp.min(jnp.where(b == m, col, nblk), axis=1,
                          keepdims=True)
            return jnp.where(col == idx, -1.0, b)

        bmr = lax.fori_loop(0, k - 1, rm, bmr)
        t0 = jnp.max(bmr, axis=1, keepdims=True)
        t0_ref[...] = jnp.broadcast_to(t0, t0_ref.shape)
        out_ref[...] = jnp.broadcast_to(b_ref[0, :][None, :], (B, D))

    p = p_ref[...]
    h0 = jnp.where(p >= t0_ref[:, 0:1], p, 0.0)
    out_ref[...] += jax.lax.dot_general(
        h0, w_ref[...], (((1,), (1,)), ((), ())),
        preferred_element_type=jnp.float32)


def _fixup_body(p_ref, thr_ref, cut_ref, x0_ref, dx_ref, h_ref, out_ref,
                *, B, BH):
    j = pl.program_id(0)
    p = p_ref[...]
    col = j * BH + jax.lax.broadcasted_iota(jnp.int32, (B, BH), 1)
    t = thr_ref[:, 0][:, None]
    c = cut_ref[:, 0][:, None]
    h_ref[...] = jnp.where((p > t) | ((p == t) & (col <= c)), p, 0.0)

    @pl.when(j == 0)
    def _():
        out_ref[...] = x0_ref[...] + dx_ref[...]


def _dec_exact_body(p_ref, w_ref, b_ref, thr_ref, cut_ref, h_ref, out_ref,
                    *, B, D, BH):
    j = pl.program_id(0)
    p = p_ref[...]
    col = j * BH + jax.lax.broadcasted_iota(jnp.int32, (B, BH), 1)
    t = thr_ref[:, 0][:, None]
    c = cut_ref[:, 0][:, None]
    h = jnp.where((p > t) | ((p == t) & (col <= c)), p, 0.0)
    h_ref[...] = h

    @pl.when(j == 0)
    def _():
        out_ref[...] = jnp.broadcast_to(b_ref[0, :][None, :], (B, D))

    out_ref[...] += jax.lax.dot_general(
        h, w_ref[...], (((1,), (1,)), ((), ())),
        preferred_element_type=jnp.float32)


def kernel(x, W_enc, b_enc, W_dec, b_dec):
    B, D = x.shape
    H = W_enc.shape[0]
    k = max(0, min(K_TOP, H))
    BH = 1024
    nsteps = H // BH

    pre, bm = pl.pallas_call(
        functools.partial(_enc_body, B=B, BH=BH),
        grid=(nsteps,),
        in_specs=[
            pl.BlockSpec((B, D), lambda j: (0, 0)),
            pl.BlockSpec((BH, D), lambda j: (j, 0)),
            pl.BlockSpec((1, BH), lambda j: (0, j)),
        ],
        out_specs=[
            pl.BlockSpec((B, BH), lambda j: (0, j)),
            pl.BlockSpec((1, B, BH // _BLK), lambda j: (j, 0, 0)),
        ],
        out_shape=[
            jax.ShapeDtypeStruct((B, H), jnp.float32),
            jax.ShapeDtypeStruct((nsteps, B, BH // _BLK), jnp.float32),
        ],
    )(x, W_enc, b_enc.reshape(1, H))

    thr, cut, dx = _sc_stage(pre, bm, W_dec.reshape(-1), B=B, H=H, D=D, k=k)

    x0 = pl.pallas_call(
        functools.partial(_dec_guess_body, B=B, D=D, BH=BH, k=k,
                          nsteps=nsteps),
        grid=(nsteps,),
        in_specs=[
            pl.BlockSpec((B, BH), lambda j: (0, j)),
            pl.BlockSpec((D, BH), lambda j: (0, j)),
            pl.BlockSpec((1, D), lambda j: (0, 0)),
            pl.BlockSpec((nsteps, B, BH // _BLK), lambda j: (0, 0, 0)),
        ],
        out_specs=pl.BlockSpec((B, D), lambda j: (0, 0)),
        out_shape=jax.ShapeDtypeStruct((B, D), jnp.float32),
        scratch_shapes=[pltpu.VMEM((B, _BLK), jnp.float32)],
    )(pre, W_dec, b_dec.reshape(1, D), bm)

    def fast(pre, W_dec, b_dec2, thr, cut, x0, dx):
        return pl.pallas_call(
            functools.partial(_fixup_body, B=B, BH=BH),
            grid=(nsteps,),
            in_specs=[
                pl.BlockSpec((B, BH), lambda j: (0, j)),
                pl.BlockSpec((B, _L), lambda j: (0, 0)),
                pl.BlockSpec((B, _L), lambda j: (0, 0)),
                pl.BlockSpec((B, D), lambda j: (0, 0)),
                pl.BlockSpec((B, D), lambda j: (0, 0)),
            ],
            out_specs=[
                pl.BlockSpec((B, BH), lambda j: (0, j)),
                pl.BlockSpec((B, D), lambda j: (0, 0)),
            ],
            out_shape=[
                jax.ShapeDtypeStruct((B, H), jnp.float32),
                jax.ShapeDtypeStruct((B, D), jnp.float32),
            ],
        )(pre, thr, cut, x0, dx)

    def slow(pre, W_dec, b_dec2, thr, cut, x0, dx):
        return pl.pallas_call(
            functools.partial(_dec_exact_body, B=B, D=D, BH=BH),
            grid=(nsteps,),
            in_specs=[
                pl.BlockSpec((B, BH), lambda j: (0, j)),
                pl.BlockSpec((D, BH), lambda j: (0, j)),
                pl.BlockSpec((1, D), lambda j: (0, 0)),
                pl.BlockSpec((B, _L), lambda j: (0, 0)),
                pl.BlockSpec((B, _L), lambda j: (0, 0)),
            ],
            out_specs=[
                pl.BlockSpec((B, BH), lambda j: (0, j)),
                pl.BlockSpec((B, D), lambda j: (0, 0)),
            ],
            out_shape=[
                jax.ShapeDtypeStruct((B, H), jnp.float32),
                jax.ShapeDtypeStruct((B, D), jnp.float32),
            ],
        )(pre, W_dec, b_dec2, thr, cut)

    overflow = jnp.any(cut[:, 1] > 0)
    h, x_hat = lax.cond(overflow, slow, fast,
                        pre, W_dec, b_dec.reshape(1, D), thr, cut, x0, dx)
    return (h, x_hat)


# speedup vs baseline: 1.8449x; 1.8449x over previous
import functools

import jax
import jax.numpy as jnp
from jax import lax
from jax.experimental import pallas as pl
from jax.experimental.pallas import tpu as pltpu
from jax.experimental.pallas import tpu_sc as plsc

K_TOP = 32
_L = 16
_CH = 16
_NC = 2
_NS = 16


def _enc_body(x_ref, w_ref, b_ref, out_ref):
    acc = jax.lax.dot_general(
        x_ref[...], w_ref[...], (((1,), (1,)), ((), ())),
        preferred_element_type=jnp.float32)
    out_ref[...] = jnp.maximum(acc + b_ref[0, :][None, :], 0.0)


def _topk_body(pre_ref, h_ref, work_ref, *, B, H, k):
    work_ref[...] = pre_ref[...]
    col = jax.lax.broadcasted_iota(jnp.int32, (B, H), 1)

    def step(_, carry):
        w = work_ref[...]
        m = jnp.max(w, axis=1, keepdims=True)
        idx = jnp.min(jnp.where(w == m, col, H), axis=1, keepdims=True)
        work_ref[...] = jnp.where(col == idx, -jnp.inf, w)
        return carry

    jax.lax.fori_loop(0, k, step, 0)
    h_ref[...] = jnp.where(work_ref[...] == -jnp.inf, pre_ref[...], 0.0)


def _topk_sc_body(pre_hbm, h_hbm, row_v, h_v, chunkv, chunki, sem, *, H, k,
                  num_cores):
    wid = lax.axis_index("s") * num_cores + lax.axis_index("c")
    cp = pltpu.async_copy(pre_hbm.at[wid], row_v, sem)

    zeros16 = jnp.zeros((_L,), jnp.float32)

    def zbody(j, c):
        h_v[pl.ds(j * _L, _L)] = zeros16
        return c

    lax.fori_loop(0, H // _L, zbody, 0)
    cp.wait()

    iota16 = lax.iota(jnp.int32, _L)
    csz = H // _CH
    nv = csz // _L

    for c in range(_CH):
        base = c * csz

        def p1(j, carry, base=base):
            bv, bi = carry
            off = base + j * _L
            v = row_v[pl.ds(off, _L)]
            take = v > bv
            return (jnp.where(take, v, bv),
                    jnp.where(take, off + iota16, bi))

        bv, bi = lax.fori_loop(
            1, nv, p1, (row_v[pl.ds(base, _L)], base + iota16))
        chunkv[pl.ds(c * _L, _L)] = bv
        chunki[pl.ds(c * _L, _L)] = bi

    big = jnp.int32(1 << 30)
    lane0 = iota16 == 0

    def _perm(v, perm):
        return v.at[perm].get(mode="promise_in_bounds")

    def _bfly(v, op):
        for s in (8, 4, 2, 1):
            v = op(v, _perm(v, iota16 ^ s))
        return v

    def emit(t, carry):
        bv = chunkv[pl.ds(0, _L)]
        bi = chunki[pl.ds(0, _L)]
        for c in range(1, _CH):
            v = chunkv[pl.ds(c * _L, _L)]
            i = chunki[pl.ds(c * _L, _L)]
            take = (v > bv) | ((v == bv) & (i < bi))
            bv = jnp.where(take, v, bv)
            bi = jnp.where(take, i, bi)
        m = _bfly(bv, jnp.maximum)
        idx = _bfly(jnp.where(bv == m, bi, big), jnp.minimum)
        plsc.store_scatter(h_v, [idx], m, mask=lane0)
        plsc.store_scatter(row_v, [idx], jnp.full((_L,), -1.0, jnp.float32),
                           mask=lane0)
        base = (idx // csz) * csz + idx % _L
        gv = gi = None
        for g in range(nv // _L):
            ii = base + _L * (iota16 + _L * g)
            vv = plsc.load_gather(row_v, [ii])
            if gv is None:
                gv, gi = vv, ii
            else:
                take = (vv > gv) | ((vv == gv) & (ii < gi))
                gv = jnp.where(take, vv, gv)
                gi = jnp.where(take, ii, gi)
        m2 = _bfly(gv, jnp.maximum)
        i2 = _bfly(jnp.where(gv == m2, gi, big), jnp.minimum)
        pos = (idx // csz) * _L + idx % _L
        plsc.store_scatter(chunkv, [pos], m2, mask=lane0)
        plsc.store_scatter(chunki, [pos], i2, mask=lane0)
        return carry

    lax.fori_loop(0, k, emit, 0)
    pltpu.sync_copy(h_v, h_hbm.at[wid])


def _topk_sc(pre, *, B, H, k):
    mesh = plsc.VectorSubcoreMesh(core_axis_name="c", subcore_axis_name="s",
                                  num_cores=_NC, num_subcores=_NS)
    return pl.kernel(
        functools.partial(_topk_sc_body, H=H, k=k, num_cores=_NC),
        out_type=jax.ShapeDtypeStruct((B, H), jnp.float32),
        mesh=mesh,
        compiler_params=pltpu.CompilerParams(
            needs_layout_passes=False, disable_bounds_checks=True,
            skip_device_barrier=True),
        scratch_types=[
            pltpu.VMEM((H,), jnp.float32),
            pltpu.VMEM((H,), jnp.float32),
            pltpu.VMEM((_CH * _L,), jnp.float32),
            pltpu.VMEM((_CH * _L,), jnp.int32),
            pltpu.SemaphoreType.DMA,
        ],
    )(pre)


def _dec_body(h_ref, w_ref, b_ref, out_ref, *, B, D):
    @pl.when(pl.program_id(0) == 0)
    def _():
        out_ref[...] = jnp.broadcast_to(b_ref[0, :][None, :], (B, D))

    out_ref[...] += jax.lax.dot_general(
        h_ref[...], w_ref[...], (((1,), (1,)), ((), ())),
        preferred_element_type=jnp.float32)


def kernel(x, W_enc, b_enc, W_dec, b_dec):
    B, D = x.shape
    H = W_enc.shape[0]
    k = max(0, min(K_TOP, H))
    BH = 1024

    pre = pl.pallas_call(
        _enc_body,
        grid=(H // BH,),
        in_specs=[
            pl.BlockSpec((B, D), lambda j: (0, 0)),
            pl.BlockSpec((BH, D), lambda j: (j, 0)),
            pl.BlockSpec((1, BH), lambda j: (0, j)),
        ],
        out_specs=pl.BlockSpec((B, BH), lambda j: (0, j)),
        out_shape=jax.ShapeDtypeStruct((B, H), jnp.float32),
    )(x, W_enc, b_enc.reshape(1, H))

    h = _topk_sc(pre, B=B, H=H, k=k)

    x_hat = pl.pallas_call(
        functools.partial(_dec_body, B=B, D=D),
        grid=(H // BH,),
        in_specs=[
            pl.BlockSpec((B, BH), lambda j: (0, j)),
            pl.BlockSpec((D, BH), lambda j: (0, j)),
            pl.BlockSpec((1, D), lambda j: (0, 0)),
        ],
        out_specs=pl.BlockSpec((B, D), lambda j: (0, 0)),
        out_shape=jax.ShapeDtypeStruct((B, D), jnp.float32),
    )(h, W_dec, b_dec.reshape(1, D))

    return (h, x_hat)


# speedup vs baseline: 1.8551x; 1.0056x over previous
import functools

import jax
import jax.numpy as jnp
from jax import lax
from jax.experimental import pallas as pl
from jax.experimental.pallas import tpu as pltpu
from jax.experimental.pallas import tpu_sc as plsc

K_TOP = 32
_L = 16
_CH = 16
_NC = 2
_NS = 16


def _enc_body(x_ref, w_ref, b_ref, out_ref):
    acc = jax.lax.dot_general(
        x_ref[...], w_ref[...], (((1,), (1,)), ((), ())),
        preferred_element_type=jnp.float32)
    out_ref[...] = jnp.maximum(acc + b_ref[0, :][None, :], 0.0)


def _topk_sc_body(pre_hbm, h_hbm, row_v, h_v, chunkv, chunki, sem, *, H, k,
                  num_cores):
    wid = lax.axis_index("s") * num_cores + lax.axis_index("c")
    cp = pltpu.async_copy(pre_hbm.at[wid], row_v, sem)

    zeros16 = jnp.zeros((_L,), jnp.float32)

    def zbody(j, c):
        h_v[pl.ds(j * _L, _L)] = zeros16
        return c

    lax.fori_loop(0, H // _L, zbody, 0)
    cp.wait()

    iota16 = lax.iota(jnp.int32, _L)
    csz = H // _CH
    nv = csz // _L

    for c in range(_CH):
        base = c * csz

        def p1(j, carry, base=base):
            bv, bi = carry
            off = base + j * _L
            v = row_v[pl.ds(off, _L)]
            take = v > bv
            return (jnp.where(take, v, bv),
                    jnp.where(take, off + iota16, bi))

        bv, bi = lax.fori_loop(
            1, nv, p1, (row_v[pl.ds(base, _L)], base + iota16))
        chunkv[pl.ds(c * _L, _L)] = bv
        chunki[pl.ds(c * _L, _L)] = bi

    big = jnp.int32(1 << 30)
    lane0 = iota16 == 0

    def _perm(v, perm):
        return v.at[perm].get(mode="promise_in_bounds")

    def _bfly(v, op):
        for s in (8, 4, 2, 1):
            v = op(v, _perm(v, iota16 ^ s))
        return v

    def emit(t, carry):
        bv = chunkv[pl.ds(0, _L)]
        bi = chunki[pl.ds(0, _L)]
        for c in range(1, _CH):
            v = chunkv[pl.ds(c * _L, _L)]
            i = chunki[pl.ds(c * _L, _L)]
            take = (v > bv) | ((v == bv) & (i < bi))
            bv = jnp.where(take, v, bv)
            bi = jnp.where(take, i, bi)
        m = _bfly(bv, jnp.maximum)
        idx = _bfly(jnp.where(bv == m, bi, big), jnp.minimum)
        plsc.store_scatter(h_v, [idx], m, mask=lane0)
        plsc.store_scatter(row_v, [idx], jnp.full((_L,), -1.0, jnp.float32),
                           mask=lane0)
        base = (idx // csz) * csz + idx % _L
        gv = gi = None
        for g in range(nv // _L):
            ii = base + _L * (iota16 + _L * g)
            vv = plsc.load_gather(row_v, [ii])
            if gv is None:
                gv, gi = vv, ii
            else:
                take = (vv > gv) | ((vv == gv) & (ii < gi))
                gv = jnp.where(take, vv, gv)
                gi = jnp.where(take, ii, gi)
        m2 = _bfly(gv, jnp.maximum)
        i2 = _bfly(jnp.where(gv == m2, gi, big), jnp.minimum)
        pos = (idx // csz) * _L + idx % _L
        plsc.store_scatter(chunkv, [pos], m2, mask=lane0)
        plsc.store_scatter(chunki, [pos], i2, mask=lane0)
        return carry

    lax.fori_loop(0, k, emit, 0)
    pltpu.sync_copy(h_v, h_hbm.at[wid])


def _topk_sc(pre, *, B, H, k):
    mesh = plsc.VectorSubcoreMesh(core_axis_name="c", subcore_axis_name="s",
                                  num_cores=_NC, num_subcores=_NS)
    return pl.kernel(
        functools.partial(_topk_sc_body, H=H, k=k, num_cores=_NC),
        out_type=jax.ShapeDtypeStruct((B, H), jnp.float32),
        mesh=mesh,
        compiler_params=pltpu.CompilerParams(
            needs_layout_passes=False, disable_bounds_checks=True,
            skip_device_barrier=True),
        scratch_types=[
            pltpu.VMEM((H,), jnp.float32),
            pltpu.VMEM((H,), jnp.float32),
            pltpu.VMEM((_CH * _L,), jnp.float32),
            pltpu.VMEM((_CH * _L,), jnp.int32),
            pltpu.SemaphoreType.DMA,
        ],
    )(pre)


def _dec_body(h_ref, w_ref, b_ref, out_ref, *, B, D):
    @pl.when(pl.program_id(0) == 0)
    def _():
        out_ref[...] = jnp.broadcast_to(b_ref[0, :][None, :], (B, D))

    out_ref[...] += jax.lax.dot_general(
        h_ref[...], w_ref[...], (((1,), (1,)), ((), ())),
        preferred_element_type=jnp.float32)


def kernel(x, W_enc, b_enc, W_dec, b_dec):
    B, D = x.shape
    H = W_enc.shape[0]
    k = max(0, min(K_TOP, H))
    BH = 1024

    pre = pl.pallas_call(
        _enc_body,
        grid=(H // BH,),
        in_specs=[
            pl.BlockSpec((B, D), lambda j: (0, 0)),
            pl.BlockSpec((BH, D), lambda j: (j, 0)),
            pl.BlockSpec((1, BH), lambda j: (0, j)),
        ],
        out_specs=pl.BlockSpec((B, BH), lambda j: (0, j)),
        out_shape=jax.ShapeDtypeStruct((B, H), jnp.float32),
    )(x, W_enc, b_enc.reshape(1, H))

    h = _topk_sc(pre, B=B, H=H, k=k)

    x_hat = pl.pallas_call(
        functools.partial(_dec_body, B=B, D=D),
        grid=(H // BH,),
        in_specs=[
            pl.BlockSpec((B, BH), lambda j: (0, j)),
            pl.BlockSpec((D, BH), lambda j: (0, j)),
            pl.BlockSpec((1, D), lambda j: (0, 0)),
        ],
        out_specs=pl.BlockSpec((B, D), lambda j: (0, 0)),
        out_shape=jax.ShapeDtypeStruct((B, D), jnp.float32),
    )(h, W_dec, b_dec.reshape(1, D))

    return (h, x_hat)
